# baseline (speedup 1.0000x reference)
import jax
import jax.numpy as jnp
from jax.experimental import pallas as pl
from jax.experimental.pallas import tpu as pltpu

_ROWS = 2000


def _fold_block(x_bf, seg, logits, out_ref, m_ref, s_ref):
    nseg = out_ref.shape[1]
    lane = jax.lax.broadcasted_iota(jnp.int32, (seg.shape[0], nseg), 1)
    masked = jnp.where(seg == lane, logits, jnp.float32(-3e38))

    bmax = jnp.max(masked, axis=0, keepdims=True)
    m_old = m_ref[...]
    m_new = jnp.maximum(m_old, bmax)
    rescale = jnp.exp(m_old - m_new)
    p = jnp.exp(masked - m_new)

    m_ref[...] = m_new
    s_ref[...] = s_ref[...] * rescale + jnp.sum(p, axis=0, keepdims=True)
    contrib = jax.lax.dot_general(
        x_bf, p.astype(jnp.bfloat16),
        dimension_numbers=(((0,), (0,)), ((), ())),
        preferred_element_type=jnp.float32)
    out_ref[...] = out_ref[...] * rescale + contrib


def _mlp_logits(x_bf, w1_ref, b1_ref, w2_ref, b2_ref):
    h = jnp.tanh(jnp.dot(x_bf, w1_ref[...],
                         preferred_element_type=jnp.float32) + b1_ref[...])
    return jnp.dot(h.astype(jnp.bfloat16), w2_ref[...],
                   preferred_element_type=jnp.float32) + b2_ref[...]


def _fused_kernel(xa_ref, xb_ref, sega_ref, segb_ref,
                  w1_ref, b1_ref, w2_ref, b2_ref,
                  out_ref, m_ref, s_ref):
    i = pl.program_id(0)
    nb = pl.num_programs(0)

    @pl.when(i == 0)
    def _init():
        m_ref[...] = jnp.full(m_ref.shape, -1e30, jnp.float32)
        s_ref[...] = jnp.zeros(s_ref.shape, jnp.float32)
        out_ref[...] = jnp.zeros(out_ref.shape, jnp.float32)

    xa = xa_ref[...].astype(jnp.bfloat16)
    la = _mlp_logits(xa, w1_ref, b1_ref, w2_ref, b2_ref)
    _fold_block(xa, sega_ref[...], la, out_ref, m_ref, s_ref)

    xb = xb_ref[...].astype(jnp.bfloat16)
    lb = _mlp_logits(xb, w1_ref, b1_ref, w2_ref, b2_ref)
    _fold_block(xb, segb_ref[...], lb, out_ref, m_ref, s_ref)

    @pl.when(i == nb - 1)
    def _final():
        out_ref[...] = out_ref[...] / (s_ref[...] + 1e-8)


def kernel(x, batch, W1, b1, W2, b2):
    n, d = x.shape
    hidden = W1.shape[1]
    nseg = 64
    rows = _ROWS
    assert n % (2 * rows) == 0
    grid = n // (2 * rows)
    half = grid

    batch2 = batch.reshape(n, 1)
    out_t = pl.pallas_call(
        _fused_kernel,
        grid=(grid,),
        in_specs=[
            pl.BlockSpec((rows, d), lambda i: (i, 0)),
            pl.BlockSpec((rows, d), lambda i: (i + half, 0)),
            pl.BlockSpec((rows, 1), lambda i: (i, 0)),
            pl.BlockSpec((rows, 1), lambda i: (i + half, 0)),
            pl.BlockSpec((d, hidden), lambda i: (0, 0)),
            pl.BlockSpec((1, hidden), lambda i: (0, 0)),
            pl.BlockSpec((hidden, 1), lambda i: (0, 0)),
            pl.BlockSpec((1, 1), lambda i: (0, 0)),
        ],
        out_specs=pl.BlockSpec((d, nseg), lambda i: (0, 0)),
        out_shape=jax.ShapeDtypeStruct((d, nseg), jnp.float32),
        scratch_shapes=[
            pltpu.VMEM((1, nseg), jnp.float32),
            pltpu.VMEM((1, nseg), jnp.float32),
        ],
    )(x, x, batch2, batch2, W1.astype(jnp.bfloat16),
      b1.reshape(1, hidden), W2.astype(jnp.bfloat16), b2.reshape(1, 1))
    return out_t.T


# speedup vs baseline: 1.0640x; 1.0640x over previous
import jax
import jax.numpy as jnp
from jax.experimental import pallas as pl
from jax.experimental.pallas import tpu as pltpu

_ROWS = 5000


def _fused_kernel(x_ref, lo_ref, hi_ref, w1_ref, b1_ref, w2_ref, b2_ref,
                  out_ref, m_ref, s_ref):
    i = pl.program_id(0)
    nb = pl.num_programs(0)
    nseg = out_ref.shape[1]
    rows = x_ref.shape[0]

    @pl.when(i == 0)
    def _init():
        m_ref[...] = jnp.full(m_ref.shape, -1e30, jnp.float32)
        s_ref[...] = jnp.zeros(s_ref.shape, jnp.float32)
        out_ref[...] = jnp.zeros(out_ref.shape, jnp.float32)

    x = x_ref[...].astype(jnp.bfloat16)
    h = jnp.tanh(jnp.dot(x, w1_ref[...],
                         preferred_element_type=jnp.float32) + b1_ref[...])
    logits = jnp.dot(h.astype(jnp.bfloat16), w2_ref[...],
                     preferred_element_type=jnp.float32) + b2_ref[...]

    gidx = i * rows + jax.lax.broadcasted_iota(jnp.int32, (rows, nseg), 0)
    inseg = (gidx >= lo_ref[...]) & (gidx < hi_ref[...])
    masked = jnp.where(inseg, logits, jnp.float32(-3e38))

    bmax = jnp.max(masked, axis=0, keepdims=True)
    m_old = m_ref[...]
    m_new = jnp.maximum(m_old, bmax)
    rescale = jnp.exp(m_old - m_new)
    p = jnp.exp(masked - m_new)

    m_ref[...] = m_new
    s_ref[...] = s_ref[...] * rescale + jnp.sum(p, axis=0, keepdims=True)
    contrib = jax.lax.dot_general(
        x, p.astype(jnp.bfloat16),
        dimension_numbers=(((0,), (0,)), ((), ())),
        preferred_element_type=jnp.float32)
    out_ref[...] = out_ref[...] * rescale + contrib

    @pl.when(i == nb - 1)
    def _final():
        out_ref[...] = out_ref[...] / (s_ref[...] + 1e-8)


def kernel(x, batch, W1, b1, W2, b2):
    n, d = x.shape
    hidden = W1.shape[1]
    nseg = 64
    rows = _ROWS
    assert n % rows == 0
    grid = n // rows

    bounds = jnp.searchsorted(
        batch, jnp.arange(nseg + 1, dtype=batch.dtype)).astype(jnp.int32)
    lo = bounds[:nseg].reshape(1, nseg)
    hi = bounds[1:].reshape(1, nseg)

    out_t = pl.pallas_call(
        _fused_kernel,
        grid=(grid,),
        in_specs=[
            pl.BlockSpec((rows, d), lambda i: (i, 0)),
            pl.BlockSpec((1, nseg), lambda i: (0, 0)),
            pl.BlockSpec((1, nseg), lambda i: (0, 0)),
            pl.BlockSpec((d, hidden), lambda i: (0, 0)),
            pl.BlockSpec((1, hidden), lambda i: (0, 0)),
            pl.BlockSpec((hidden, 1), lambda i: (0, 0)),
            pl.BlockSpec((1, 1), lambda i: (0, 0)),
        ],
        out_specs=pl.BlockSpec((d, nseg), lambda i: (0, 0)),
        out_shape=jax.ShapeDtypeStruct((d, nseg), jnp.float32),
        scratch_shapes=[
            pltpu.VMEM((1, nseg), jnp.float32),
            pltpu.VMEM((1, nseg), jnp.float32),
        ],
    )(x, lo, hi, W1.astype(jnp.bfloat16),
      b1.reshape(1, hidden), W2.astype(jnp.bfloat16), b2.reshape(1, 1))
    return out_t.T


# speedup vs baseline: 1.1073x; 1.0406x over previous
import jax
import jax.numpy as jnp
from jax.experimental import pallas as pl
from jax.experimental.pallas import tpu as pltpu

_ROWS = 5000


def _fused_kernel(x_ref, lo_ref, hi_ref, w1_ref, b1_ref, w2_ref, b2_ref,
                  out_ref, m_ref, s_ref):
    i = pl.program_id(0)
    nb = pl.num_programs(0)
    nseg = out_ref.shape[1]
    rows = x_ref.shape[0]

    @pl.when(i == 0)
    def _init():
        m_ref[...] = jnp.full(m_ref.shape, -1e30, jnp.float32)
        s_ref[...] = jnp.zeros(s_ref.shape, jnp.float32)
        out_ref[...] = jnp.zeros(out_ref.shape, jnp.float32)

    x = x_ref[...].astype(jnp.bfloat16)
    h = jnp.tanh(jnp.dot(x, w1_ref[...],
                         preferred_element_type=jnp.float32) + b1_ref[...])
    logits = jnp.dot(h.astype(jnp.bfloat16), w2_ref[...],
                     preferred_element_type=jnp.float32) + b2_ref[...]

    gidx = i * rows + jax.lax.broadcasted_iota(jnp.int32, (rows, nseg), 0)
    inseg = (gidx >= lo_ref[...]) & (gidx < hi_ref[...])
    masked = jnp.where(inseg, logits, jnp.float32(-3e38))

    bmax = jnp.max(masked, axis=0, keepdims=True)
    m_old = m_ref[...]
    m_new = jnp.maximum(m_old, bmax)
    rescale = jnp.exp(m_old - m_new)
    p = jnp.exp(masked - m_new)

    m_ref[...] = m_new
    s_ref[...] = s_ref[...] * rescale + jnp.sum(p, axis=0, keepdims=True)
    contrib = jax.lax.dot_general(
        x, p.astype(jnp.bfloat16),
        dimension_numbers=(((0,), (0,)), ((), ())),
        preferred_element_type=jnp.float32)
    out_ref[...] = out_ref[...] * rescale + contrib

    @pl.when(i == nb - 1)
    def _final():
        out_ref[...] = out_ref[...] / (s_ref[...] + 1e-8)


def kernel(x, batch, W1, b1, W2, b2):
    n, d = x.shape
    hidden = W1.shape[1]
    nseg = 64
    rows = _ROWS
    assert n % rows == 0
    grid = n // rows

    counts = jnp.zeros((nseg,), jnp.int32).at[batch].add(1)
    bounds = jnp.concatenate(
        [jnp.zeros((1,), jnp.int32), jnp.cumsum(counts, dtype=jnp.int32)])
    lo = bounds[:nseg].reshape(1, nseg)
    hi = bounds[1:].reshape(1, nseg)

    out_t = pl.pallas_call(
        _fused_kernel,
        grid=(grid,),
        in_specs=[
            pl.BlockSpec((rows, d), lambda i: (i, 0)),
            pl.BlockSpec((1, nseg), lambda i: (0, 0)),
            pl.BlockSpec((1, nseg), lambda i: (0, 0)),
            pl.BlockSpec((d, hidden), lambda i: (0, 0)),
            pl.BlockSpec((1, hidden), lambda i: (0, 0)),
            pl.BlockSpec((hidden, 1), lambda i: (0, 0)),
            pl.BlockSpec((1, 1), lambda i: (0, 0)),
        ],
        out_specs=pl.BlockSpec((d, nseg), lambda i: (0, 0)),
        out_shape=jax.ShapeDtypeStruct((d, nseg), jnp.float32),
        scratch_shapes=[
            pltpu.VMEM((1, nseg), jnp.float32),
            pltpu.VMEM((1, nseg), jnp.float32),
        ],
    )(x, lo, hi, W1.astype(jnp.bfloat16),
      b1.reshape(1, hidden), W2.astype(jnp.bfloat16), b2.reshape(1, 1))
    return out_t.T


# speedup vs baseline: 1.2941x; 1.1688x over previous
import jax
import jax.numpy as jnp
from jax.experimental import pallas as pl
from jax.experimental.pallas import tpu as pltpu

_ROWS = 5000
_HCHUNK = 10000


def _fused_kernel(x_ref, ids_ref, w1_ref, b1_ref, w2_ref, b2_ref,
                  out_ref, m_ref, s_ref, lo_ref, hi_ref):
    i = pl.program_id(0)
    nb = pl.num_programs(0)
    nseg = out_ref.shape[1]
    rows = x_ref.shape[0]
    n = ids_ref.shape[1]

    @pl.when(i == 0)
    def _init():
        m_ref[...] = jnp.full(m_ref.shape, -1e30, jnp.float32)
        s_ref[...] = jnp.zeros(s_ref.shape, jnp.float32)
        out_ref[...] = jnp.zeros(out_ref.shape, jnp.float32)
        seg_iota = jax.lax.broadcasted_iota(jnp.int32, (nseg, 1), 0)
        acc = jnp.zeros((nseg, 1), jnp.int32)
        for k in range(n // _HCHUNK):
            chunk = ids_ref[:, k * _HCHUNK:(k + 1) * _HCHUNK]
            acc = acc + jnp.sum((chunk < seg_iota).astype(jnp.int32),
                                axis=1, keepdims=True)
        hi64 = jnp.concatenate(
            [acc[1:, :], jnp.full((1, 1), n, jnp.int32)], axis=0)
        lo_ref[...] = acc.T
        hi_ref[...] = hi64.T

    x = x_ref[...].astype(jnp.bfloat16)
    h = jnp.tanh(jnp.dot(x, w1_ref[...],
                         preferred_element_type=jnp.float32) + b1_ref[...])
    logits = jnp.dot(h.astype(jnp.bfloat16), w2_ref[...],
                     preferred_element_type=jnp.float32) + b2_ref[...]

    gidx = i * rows + jax.lax.broadcasted_iota(jnp.int32, (rows, nseg), 0)
    inseg = (gidx >= lo_ref[...]) & (gidx < hi_ref[...])
    masked = jnp.where(inseg, logits, jnp.float32(-3e38))

    bmax = jnp.max(masked, axis=0, keepdims=True)
    m_old = m_ref[...]
    m_new = jnp.maximum(m_old, bmax)
    rescale = jnp.exp(m_old - m_new)
    p = jnp.exp(masked - m_new)

    m_ref[...] = m_new
    s_ref[...] = s_ref[...] * rescale + jnp.sum(p, axis=0, keepdims=True)
    contrib = jax.lax.dot_general(
        x, p.astype(jnp.bfloat16),
        dimension_numbers=(((0,), (0,)), ((), ())),
        preferred_element_type=jnp.float32)
    out_ref[...] = out_ref[...] * rescale + contrib

    @pl.when(i == nb - 1)
    def _final():
        out_ref[...] = out_ref[...] / (s_ref[...] + 1e-8)


def kernel(x, batch, W1, b1, W2, b2):
    n, d = x.shape
    hidden = W1.shape[1]
    nseg = 64
    rows = _ROWS
    assert n % rows == 0 and n % _HCHUNK == 0
    grid = n // rows

    out_t = pl.pallas_call(
        _fused_kernel,
        grid=(grid,),
        in_specs=[
            pl.BlockSpec((rows, d), lambda i: (i, 0)),
            pl.BlockSpec((1, n), lambda i: (0, 0)),
            pl.BlockSpec((d, hidden), lambda i: (0, 0)),
            pl.BlockSpec((1, hidden), lambda i: (0, 0)),
            pl.BlockSpec((hidden, 1), lambda i: (0, 0)),
            pl.BlockSpec((1, 1), lambda i: (0, 0)),
        ],
        out_specs=pl.BlockSpec((d, nseg), lambda i: (0, 0)),
        out_shape=jax.ShapeDtypeStruct((d, nseg), jnp.float32),
        scratch_shapes=[
            pltpu.VMEM((1, nseg), jnp.float32),
            pltpu.VMEM((1, nseg), jnp.float32),
            pltpu.VMEM((1, nseg), jnp.int32),
            pltpu.VMEM((1, nseg), jnp.int32),
        ],
    )(x, batch.reshape(1, n), W1.astype(jnp.bfloat16),
      b1.reshape(1, hidden), W2.astype(jnp.bfloat16), b2.reshape(1, 1))
    return out_t.T
